# baseline (speedup 1.0000x reference)
import functools

import jax
import jax.numpy as jnp
from jax.experimental import pallas as pl
from jax.experimental.pallas import tpu as pltpu

N_ROWS = 5000
D_IN = 12544
D_HID = 1024
BR = 1000
BK = 896
NR = N_ROWS // BR
NK = D_IN // BK


def _boxhead_body(x_ref, w1_ref, b1_ref, w2_ref, b2_ref, w3_ref, b3_ref,
                  w4_ref, b4_ref, cls_ref, box_ref, acc_ref):
    k = pl.program_id(0)
    i = pl.program_id(1)
    rows = pl.ds(i * BR, BR)

    xb = x_ref[...].astype(jnp.bfloat16)
    w1b = w1_ref[...].astype(jnp.bfloat16)
    partial = jnp.dot(xb, w1b, preferred_element_type=jnp.float32)

    @pl.when(k == 0)
    def _init():
        acc_ref[rows, :] = partial

    @pl.when(k > 0)
    def _accum():
        acc_ref[rows, :] += partial

    @pl.when(k == NK - 1)
    def _epilogue():
        h1 = jnp.maximum(acc_ref[rows, :] + b1_ref[...], 0.0)
        h1b = h1.astype(jnp.bfloat16)
        w2b = w2_ref[...].astype(jnp.bfloat16)
        h2 = jnp.maximum(
            jnp.dot(h1b, w2b, preferred_element_type=jnp.float32)
            + b2_ref[...], 0.0)
        h2b = h2.astype(jnp.bfloat16)
        cls_ref[...] = (
            jnp.dot(h2b, w3_ref[...].astype(jnp.bfloat16),
                    preferred_element_type=jnp.float32) + b3_ref[...])
        box_ref[...] = (
            jnp.dot(h2b, w4_ref[...].astype(jnp.bfloat16),
                    preferred_element_type=jnp.float32) + b4_ref[...])


@functools.partial(jax.jit, static_argnames=())
def kernel(feature_vectors, W1, b1, W2, b2, W3, b3, W4, b4):
    c1 = b3.shape[0]
    c4 = b4.shape[0]
    grid = (NK, NR)
    out = pl.pallas_call(
        _boxhead_body,
        grid=grid,
        in_specs=[
            pl.BlockSpec((BR, BK), lambda k, i: (i, k)),
            pl.BlockSpec((BK, D_HID), lambda k, i: (k, 0)),
            pl.BlockSpec((1, D_HID), lambda k, i: (0, 0)),
            pl.BlockSpec((D_HID, D_HID), lambda k, i: (0, 0)),
            pl.BlockSpec((1, D_HID), lambda k, i: (0, 0)),
            pl.BlockSpec((D_HID, c1), lambda k, i: (0, 0)),
            pl.BlockSpec((1, c1), lambda k, i: (0, 0)),
            pl.BlockSpec((D_HID, c4), lambda k, i: (0, 0)),
            pl.BlockSpec((1, c4), lambda k, i: (0, 0)),
        ],
        out_specs=[
            pl.BlockSpec((BR, c1), lambda k, i: (i, 0)),
            pl.BlockSpec((BR, c4), lambda k, i: (i, 0)),
        ],
        out_shape=[
            jax.ShapeDtypeStruct((N_ROWS, c1), jnp.float32),
            jax.ShapeDtypeStruct((N_ROWS, c4), jnp.float32),
        ],
        scratch_shapes=[pltpu.VMEM((N_ROWS, D_HID), jnp.float32)],
        compiler_params=pltpu.CompilerParams(
            dimension_semantics=("arbitrary", "arbitrary"),
        ),
    )(feature_vectors, W1, b1.reshape(1, -1), W2, b2.reshape(1, -1),
      W3, b3.reshape(1, -1), W4, b4.reshape(1, -1))
    return (out[0], out[1])
